# baseline (speedup 1.0000x reference)
import numpy as np
import jax
import jax.numpy as jnp
from jax.experimental import pallas as pl
from jax.experimental.pallas import tpu as pltpu


def _toeplitz_rows(base, nrows, width):
    lead = base.shape[:-1]
    per = width + nrows
    c = jnp.pad(base, [(0, 0)] * len(lead) + [(0, nrows)])
    t = jnp.tile(c, (1,) * len(lead) + (nrows,))
    y = t[..., :nrows * (per - 1)].reshape(*lead, nrows, per - 1)
    return y[..., :width]


def _build_w1_band(w1):
    k = jnp.pad(w1.reshape(10, 5, 5), ((0, 0), (0, 0), (0, 23)))
    base = jnp.stack([
        jnp.pad(k, ((0, 0), (0, 1), (0, 0))),
        jnp.pad(k, ((0, 0), (1, 0), (0, 0))),
    ]).reshape(2, 10, 168)
    y = _toeplitz_rows(base, 24, 168)
    return y.reshape(2, 10, 12, 2, 168).transpose(0, 3, 1, 2, 4).reshape(480, 168)


def _build_w2_band(w2):
    k = w2.reshape(20, 10, 5, 5).transpose(0, 2, 1, 3)
    k = jnp.pad(k, ((0, 0), (0, 0), (0, 0), (0, 7))).reshape(20, 5, 120)
    base = jnp.stack([
        jnp.pad(k, ((0, 0), (0, 1), (0, 0))),
        jnp.pad(k, ((0, 0), (1, 0), (0, 0))),
    ]).reshape(2, 20, 720)
    y = _toeplitz_rows(base, 8, 720)
    return y.reshape(2, 20, 4, 2, 720).transpose(0, 3, 1, 2, 4).reshape(320, 720)


def _net_kernel(x0_ref, x1_ref, x2_ref, x3_ref, x4_ref, x5_ref, x6_ref, x7_ref,
                w1b_ref,
                b1v_ref,
                w2b_ref,
                b2v_ref,
                w320_ref,
                fb1_ref,
                fw2_ref,
                fb2_ref,
                o_ref,
                xt_ref,
                a1_ref,
                a2_ref):
    tb = o_ref.shape[1]

    xs = (x0_ref, x1_ref, x2_ref, x3_ref, x4_ref, x5_ref, x6_ref, x7_ref)
    nsub = tb // 8
    for i in range(8):
        for g in range(7):
            v = jnp.concatenate([xs[i][:, 4 * g + j, :] for j in range(4)],
                                axis=1)
            xt_ref[112 * g:112 * g + 112, nsub * i:nsub * (i + 1)] = v.T

    w1b = w1b_ref[...]
    b1v = b1v_ref[...]
    for hp in range(12):
        rhs = xt_ref[56 * hp:56 * hp + 168]
        r = jnp.dot(w1b, rhs, preferred_element_type=jnp.float32)
        m = jnp.maximum(r[:240], r[240:])
        m = jnp.maximum(m[:120], m[120:])
        a1_ref[hp] = jnp.maximum(m + b1v, 0.0)

    w2b = w2b_ref[...]
    b2v = b2v_ref[...]
    for p in range(4):
        rhs = a1_ref[2 * p:2 * p + 6].reshape(720, tb)
        r = jnp.dot(w2b, rhs, preferred_element_type=jnp.float32)
        m = jnp.maximum(r[:160], r[160:])
        m = jnp.maximum(m[:80], m[80:])
        a2_ref[p] = jnp.maximum(m + b2v, 0.0)

    h = jnp.dot(w320_ref[...], a2_ref[...].reshape(320, tb),
                preferred_element_type=jnp.float32)
    h = jnp.maximum(h + fb1_ref[...], 0.0)
    logits = jnp.dot(fw2_ref[...], h, preferred_element_type=jnp.float32)
    logits = logits + fb2_ref[...]
    zc = logits - jnp.max(logits, axis=0, keepdims=True)
    o_ref[...] = zc - jnp.log(jnp.sum(jnp.exp(zc), axis=0, keepdims=True))


def kernel(x_nchw, w1, b1, w2, b2, fw1, fb1, fw2, fb2):
    n = x_nchw.shape[0]
    npad = ((n + 127) // 128) * 128
    tb = next(t for t in (1024, 512, 256, 128) if npad % t == 0)

    x3d = x_nchw.reshape(n, 28, 28)
    if npad != n:
        x3d = jnp.pad(x3d, ((0, npad - n), (0, 0), (0, 0)))

    w1b = _build_w1_band(w1)
    w2b = _build_w2_band(w2)
    b1v = jnp.repeat(b1, 12).reshape(120, 1)
    b2v = jnp.repeat(b2, 4).reshape(80, 1)
    w320 = fw1.reshape(4, 4, 50, 20).transpose(2, 0, 3, 1).reshape(50, 320)

    out = pl.pallas_call(
        _net_kernel,
        out_shape=jax.ShapeDtypeStruct((10, npad), jnp.float32),
        grid_spec=pltpu.PrefetchScalarGridSpec(
            num_scalar_prefetch=0,
            grid=(npad // tb,),
            in_specs=[
                *[pl.BlockSpec((tb // 8, 28, 28), lambda nb, i=i: (8 * nb + i, 0, 0))
                  for i in range(8)],
                pl.BlockSpec((480, 168), lambda nb: (0, 0)),
                pl.BlockSpec((120, 1), lambda nb: (0, 0)),
                pl.BlockSpec((320, 720), lambda nb: (0, 0)),
                pl.BlockSpec((80, 1), lambda nb: (0, 0)),
                pl.BlockSpec((50, 320), lambda nb: (0, 0)),
                pl.BlockSpec((50, 1), lambda nb: (0, 0)),
                pl.BlockSpec((10, 50), lambda nb: (0, 0)),
                pl.BlockSpec((10, 1), lambda nb: (0, 0)),
            ],
            out_specs=pl.BlockSpec((10, tb), lambda nb: (0, nb)),
            scratch_shapes=[
                pltpu.VMEM((784, tb), jnp.float32),
                pltpu.VMEM((12, 120, tb), jnp.float32),
                pltpu.VMEM((4, 80, tb), jnp.float32),
            ],
        ),
        compiler_params=pltpu.CompilerParams(dimension_semantics=("parallel",)),
    )(x3d, x3d, x3d, x3d, x3d, x3d, x3d, x3d, w1b, b1v, w2b, b2v, w320, fb1, fw2, fb2)

    return out[:, :n].T


# speedup vs baseline: 1.0118x; 1.0118x over previous
import numpy as np
import jax
import jax.numpy as jnp
from jax.experimental import pallas as pl
from jax.experimental.pallas import tpu as pltpu


def _toeplitz_rows(base, nrows, width):
    lead = base.shape[:-1]
    per = width + nrows
    c = jnp.pad(base, [(0, 0)] * len(lead) + [(0, nrows)])
    t = jnp.tile(c, (1,) * len(lead) + (nrows,))
    y = t[..., :nrows * (per - 1)].reshape(*lead, nrows, per - 1)
    return y[..., :width]


def _build_w1_band(w1):
    k = jnp.pad(w1.reshape(10, 5, 5), ((0, 0), (0, 0), (0, 23)))
    base = jnp.stack([
        jnp.pad(k, ((0, 0), (0, 1), (0, 0))),
        jnp.pad(k, ((0, 0), (1, 0), (0, 0))),
    ]).reshape(2, 10, 168)
    y = _toeplitz_rows(base, 24, 168)
    return y.reshape(2, 10, 12, 2, 168).transpose(0, 3, 1, 2, 4).reshape(480, 168)


def _build_w2_band(w2):
    k = w2.reshape(20, 10, 5, 5).transpose(0, 2, 1, 3)
    k = jnp.pad(k, ((0, 0), (0, 0), (0, 0), (0, 7))).reshape(20, 5, 120)
    base = jnp.stack([
        jnp.pad(k, ((0, 0), (0, 1), (0, 0))),
        jnp.pad(k, ((0, 0), (1, 0), (0, 0))),
    ]).reshape(2, 20, 720)
    y = _toeplitz_rows(base, 8, 720)
    return y.reshape(2, 20, 4, 2, 720).transpose(0, 3, 1, 2, 4).reshape(320, 720)


def _net_kernel(x0_ref, x1_ref, x2_ref, x3_ref,
                w1b_ref,
                b1v_ref,
                w2b_ref,
                b2v_ref,
                w320_ref,
                fb1_ref,
                fw2_ref,
                fb2_ref,
                o_ref,
                xt_ref,
                a1_ref,
                a2_ref):
    tb = o_ref.shape[1]

    xs = (x0_ref, x1_ref, x2_ref, x3_ref)
    nsub = tb // 4
    for i in range(4):
        for g in range(7):
            v = jnp.concatenate([xs[i][:, 4 * g + j, :] for j in range(4)],
                                axis=1)
            xt_ref[112 * g:112 * g + 112, nsub * i:nsub * (i + 1)] = v.T

    w1b = w1b_ref[...]
    b1v = b1v_ref[...]
    for hp in range(12):
        rhs = xt_ref[56 * hp:56 * hp + 168]
        r = jnp.dot(w1b, rhs, preferred_element_type=jnp.float32)
        m = jnp.maximum(r[:240], r[240:])
        m = jnp.maximum(m[:120], m[120:])
        a1_ref[hp] = jnp.maximum(m + b1v, 0.0)

    w2b = w2b_ref[...]
    b2v = b2v_ref[...]
    for p in range(4):
        rhs = a1_ref[2 * p:2 * p + 6].reshape(720, tb)
        r = jnp.dot(w2b, rhs, preferred_element_type=jnp.float32)
        m = jnp.maximum(r[:160], r[160:])
        m = jnp.maximum(m[:80], m[80:])
        a2_ref[p] = jnp.maximum(m + b2v, 0.0)

    h = jnp.dot(w320_ref[...], a2_ref[...].reshape(320, tb),
                preferred_element_type=jnp.float32)
    h = jnp.maximum(h + fb1_ref[...], 0.0)
    logits = jnp.dot(fw2_ref[...], h, preferred_element_type=jnp.float32)
    logits = logits + fb2_ref[...]
    zc = logits - jnp.max(logits, axis=0, keepdims=True)
    o_ref[...] = zc - jnp.log(jnp.sum(jnp.exp(zc), axis=0, keepdims=True))


def kernel(x_nchw, w1, b1, w2, b2, fw1, fb1, fw2, fb2):
    n = x_nchw.shape[0]
    npad = ((n + 127) // 128) * 128
    tb = next(t for t in (1024, 512, 256, 128) if npad % t == 0)

    x3d = x_nchw.reshape(n, 28, 28)
    if npad != n:
        x3d = jnp.pad(x3d, ((0, npad - n), (0, 0), (0, 0)))

    w1b = _build_w1_band(w1)
    w2b = _build_w2_band(w2)
    b1v = jnp.repeat(b1, 12).reshape(120, 1)
    b2v = jnp.repeat(b2, 4).reshape(80, 1)
    w320 = fw1.reshape(4, 4, 50, 20).transpose(2, 0, 3, 1).reshape(50, 320)

    out = pl.pallas_call(
        _net_kernel,
        out_shape=jax.ShapeDtypeStruct((10, npad), jnp.float32),
        grid_spec=pltpu.PrefetchScalarGridSpec(
            num_scalar_prefetch=0,
            grid=(npad // tb,),
            in_specs=[
                *[pl.BlockSpec((tb // 4, 28, 28), lambda nb, i=i: (4 * nb + i, 0, 0))
                  for i in range(4)],
                pl.BlockSpec((480, 168), lambda nb: (0, 0)),
                pl.BlockSpec((120, 1), lambda nb: (0, 0)),
                pl.BlockSpec((320, 720), lambda nb: (0, 0)),
                pl.BlockSpec((80, 1), lambda nb: (0, 0)),
                pl.BlockSpec((50, 320), lambda nb: (0, 0)),
                pl.BlockSpec((50, 1), lambda nb: (0, 0)),
                pl.BlockSpec((10, 50), lambda nb: (0, 0)),
                pl.BlockSpec((10, 1), lambda nb: (0, 0)),
            ],
            out_specs=pl.BlockSpec((10, tb), lambda nb: (0, nb)),
            scratch_shapes=[
                pltpu.VMEM((784, tb), jnp.float32),
                pltpu.VMEM((12, 120, tb), jnp.float32),
                pltpu.VMEM((4, 80, tb), jnp.float32),
            ],
        ),
        compiler_params=pltpu.CompilerParams(dimension_semantics=("parallel",)),
    )(x3d, x3d, x3d, x3d, w1b, b1v, w2b, b2v, w320, fb1, fw2, fb2)

    return out[:, :n].T
